# Initial kernel scaffold; baseline (speedup 1.0000x reference)
import functools

import jax
import jax.numpy as jnp
from jax import lax
from jax.experimental import pallas as pl
from jax.experimental.pallas import tpu as pltpu
from jax.experimental.pallas import tpu_sc as plsc

RES = 128
EMA_D = 0.95
THRE = 0.01
NPTS = 1048576
NCELL = RES * RES * RES
NW = 32
PPT = NPTS // NW
SLAB = NCELL // NW
CH1 = 8192
CH2 = 8192
GC = 8192
L = 16


def _wid():
    return lax.axis_index("s") * 2 + lax.axis_index("c")


@functools.partial(
    pl.kernel,
    out_type=jax.ShapeDtypeStruct((NPTS,), jnp.int32),
    mesh=plsc.VectorSubcoreMesh(core_axis_name="c", subcore_axis_name="s"),
    scratch_types=[
        pltpu.VMEM((CH1, 3), jnp.float32),
        pltpu.VMEM((CH1,), jnp.int32),
    ],
)
def _flat_idx_kernel(pts_hbm, flat_hbm, ptsb, flatb):
    w = _wid()
    base = w * PPT
    lane = lax.iota(jnp.int32, L)
    c0 = jnp.zeros((L,), jnp.int32)
    c1 = jnp.ones((L,), jnp.int32)
    c2 = jnp.full((L,), 2, jnp.int32)

    def chunk_body(ci, carry):
        off = base + ci * CH1
        pltpu.sync_copy(pts_hbm.at[pl.ds(off, CH1)], ptsb)

        def vec_body(j, carry2):
            row = j * L + lane
            x = plsc.load_gather(ptsb, [row, c0])
            y = plsc.load_gather(ptsb, [row, c1])
            z = plsc.load_gather(ptsb, [row, c2])

            def quant(t):
                tf = (t * 0.5 + 0.5) * 128.0
                ti = tf.astype(jnp.int32)
                return jnp.clip(ti, 0, RES - 1)

            f = (quant(x) * RES + quant(y)) * RES + quant(z)
            flatb[pl.ds(j * L, L)] = f
            return carry2

        lax.fori_loop(0, CH1 // L, vec_body, 0)
        pltpu.sync_copy(flatb, flat_hbm.at[pl.ds(off, CH1)])
        return carry

    lax.fori_loop(0, PPT // CH1, chunk_body, 0)


@functools.partial(
    pl.kernel,
    out_type=jax.ShapeDtypeStruct((NCELL,), jnp.float32),
    mesh=plsc.VectorSubcoreMesh(core_axis_name="c", subcore_axis_name="s"),
    scratch_types=[
        pltpu.VMEM((SLAB,), jnp.float32),
        pltpu.VMEM((CH2,), jnp.int32),
        pltpu.VMEM((CH2,), jnp.float32),
        pltpu.VMEM((GC,), jnp.float32),
        pltpu.VMEM((GC,), jnp.float32),
    ],
)
def _scatter_ema_kernel(grid_hbm, flat_hbm, val_hbm, out_hbm,
                        slab, flatb, valb, gb, ob):
    w = _wid()
    lo = w * SLAB
    neg1 = jnp.full((L,), -1.0, jnp.float32)

    def init_body(i, carry):
        slab[pl.ds(i * L, L)] = neg1
        return carry

    lax.fori_loop(0, SLAB // L, init_body, 0)

    def chunk_body(ci, carry):
        off = ci * CH2
        pltpu.sync_copy(flat_hbm.at[pl.ds(off, CH2)], flatb)
        pltpu.sync_copy(val_hbm.at[pl.ds(off, CH2)], valb)

        def vec_body(j, carry2):
            f = flatb[pl.ds(j * L, L)]
            v = valb[pl.ds(j * L, L)]
            loc = f - lo
            m = (loc >= 0) & (loc < SLAB)
            locc = jnp.clip(loc, 0, SLAB - 1)
            cnt = jnp.sum(m.astype(jnp.int32))

            @pl.when(cnt > 0)
            def _():
                g = plsc.load_gather(slab, [locc], mask=m)
                plsc.store_scatter(slab, [locc], jnp.maximum(g, v), mask=m)
                g2 = plsc.load_gather(slab, [locc], mask=m)
                need = m & (g2 < v)

                def rcond(nd):
                    return jnp.sum(nd.astype(jnp.int32)) > 0

                def rbody(nd):
                    gg = plsc.load_gather(slab, [locc], mask=nd)
                    plsc.store_scatter(slab, [locc], jnp.maximum(gg, v),
                                       mask=nd)
                    gg2 = plsc.load_gather(slab, [locc], mask=nd)
                    return nd & (gg2 < v)

                lax.while_loop(rcond, rbody, need)

            return carry2

        lax.fori_loop(0, CH2 // L, vec_body, 0)
        return carry

    lax.fori_loop(0, NPTS // CH2, chunk_body, 0)

    def gchunk_body(gi, carry):
        goff = lo + gi * GC
        pltpu.sync_copy(grid_hbm.at[pl.ds(goff, GC)], gb)

        def gvec(j, carry2):
            g = gb[pl.ds(j * L, L)]
            s = slab[pl.ds(gi * GC + j * L, L)]
            touched = s > -0.5
            new = jnp.where(touched,
                            jnp.maximum(g * EMA_D, jnp.maximum(s, 0.0)), g)
            ob[pl.ds(j * L, L)] = new
            return carry2

        lax.fori_loop(0, GC // L, gvec, 0)
        pltpu.sync_copy(ob, out_hbm.at[pl.ds(goff, GC)])
        return carry

    lax.fori_loop(0, SLAB // GC, gchunk_body, 0)


def _thresh_body(x_ref, o_ref):
    o_ref[...] = x_ref[...] > THRE


def _threshold(new_flat):
    x = new_flat.reshape(2048, 1024)
    out = pl.pallas_call(
        _thresh_body,
        out_shape=jax.ShapeDtypeStruct((2048, 1024), jnp.bool_),
        grid=(8,),
        in_specs=[pl.BlockSpec((256, 1024), lambda i: (i, 0))],
        out_specs=pl.BlockSpec((256, 1024), lambda i: (i, 0)),
    )(x)
    return out.reshape(RES, RES, RES)


def kernel(occ_val_grid, pts, val):
    grid_flat = occ_val_grid.reshape(-1)
    flat = _flat_idx_kernel(pts)
    new_flat = _scatter_ema_kernel(grid_flat, flat, val)
    occ = _threshold(new_flat)
    return new_flat.reshape(RES, RES, RES), occ


# speedup vs baseline: 1.1192x; 1.1192x over previous
import functools

import jax
import jax.numpy as jnp
from jax import lax
from jax.experimental import pallas as pl
from jax.experimental.pallas import tpu as pltpu
from jax.experimental.pallas import tpu_sc as plsc

RES = 128
EMA_D = 0.95
THRE = 0.01
NPTS = 1048576
NCELL = RES * RES * RES
NW = 32
PPT = NPTS // NW
SLAB = NCELL // NW
CH1 = 8192
CH2 = 8192
GC = 8192
L = 16


def _wid():
    return lax.axis_index("s") * 2 + lax.axis_index("c")


@functools.partial(
    pl.kernel,
    out_type=jax.ShapeDtypeStruct((NPTS,), jnp.int32),
    mesh=plsc.VectorSubcoreMesh(core_axis_name="c", subcore_axis_name="s"),
    compiler_params=pltpu.CompilerParams(needs_layout_passes=False),
    scratch_types=[
        pltpu.VMEM((CH1 * 3,), jnp.float32),
        pltpu.VMEM((CH1,), jnp.int32),
    ],
)
def _flat_idx_kernel(pts_hbm, flat_hbm, ptsb, flatb):
    w = _wid()
    base = w * PPT
    lane = lax.iota(jnp.int32, L)

    def chunk_body(ci, carry):
        off = base + ci * CH1
        pltpu.sync_copy(pts_hbm.at[pl.ds(off * 3, CH1 * 3)], ptsb)

        def vec_body(j, carry2):
            row3 = (j * L + lane) * 3
            x = plsc.load_gather(ptsb, [row3])
            y = plsc.load_gather(ptsb, [row3 + 1])
            z = plsc.load_gather(ptsb, [row3 + 2])

            def quant(t):
                tf = (t * 0.5 + 0.5) * 128.0
                ti = tf.astype(jnp.int32)
                return jnp.clip(ti, 0, RES - 1)

            f = (quant(x) * RES + quant(y)) * RES + quant(z)
            flatb[pl.ds(j * L, L)] = f
            return carry2

        lax.fori_loop(0, CH1 // L, vec_body, 0)
        pltpu.sync_copy(flatb, flat_hbm.at[pl.ds(off, CH1)])
        return carry

    lax.fori_loop(0, PPT // CH1, chunk_body, 0)


@functools.partial(
    pl.kernel,
    out_type=jax.ShapeDtypeStruct((NCELL,), jnp.float32),
    mesh=plsc.VectorSubcoreMesh(core_axis_name="c", subcore_axis_name="s"),
    compiler_params=pltpu.CompilerParams(needs_layout_passes=False),
    scratch_types=[
        pltpu.VMEM((SLAB,), jnp.float32),
        pltpu.VMEM((CH2,), jnp.int32),
        pltpu.VMEM((CH2,), jnp.float32),
        pltpu.VMEM((GC,), jnp.float32),
        pltpu.VMEM((GC,), jnp.float32),
    ],
)
def _scatter_ema_kernel(grid_hbm, flat_hbm, val_hbm, out_hbm,
                        slab, flatb, valb, gb, ob):
    w = _wid()
    lo = w * SLAB
    neg1 = jnp.full((L,), -1.0, jnp.float32)

    def init_body(i, carry):
        slab[pl.ds(i * L, L)] = neg1
        return carry

    lax.fori_loop(0, SLAB // L, init_body, 0)

    def chunk_body(ci, carry):
        off = ci * CH2
        pltpu.sync_copy(flat_hbm.at[pl.ds(off, CH2)], flatb)
        pltpu.sync_copy(val_hbm.at[pl.ds(off, CH2)], valb)

        def vec_body(j, carry2):
            f = flatb[pl.ds(j * L, L)]
            v = valb[pl.ds(j * L, L)]
            loc = f - lo
            m = (loc >= 0) & (loc < SLAB)
            locc = jnp.clip(loc, 0, SLAB - 1)
            cnt = jnp.sum(m.astype(jnp.int32))

            @pl.when(cnt > 0)
            def _():
                g = plsc.load_gather(slab, [locc], mask=m)
                plsc.store_scatter(slab, [locc], jnp.maximum(g, v), mask=m)
                g2 = plsc.load_gather(slab, [locc], mask=m)
                need = m & (g2 < v)

                def rcond(nd):
                    return jnp.sum(nd.astype(jnp.int32)) > 0

                def rbody(nd):
                    gg = plsc.load_gather(slab, [locc], mask=nd)
                    plsc.store_scatter(slab, [locc], jnp.maximum(gg, v),
                                       mask=nd)
                    gg2 = plsc.load_gather(slab, [locc], mask=nd)
                    return nd & (gg2 < v)

                lax.while_loop(rcond, rbody, need)

            return carry2

        lax.fori_loop(0, CH2 // L, vec_body, 0)
        return carry

    lax.fori_loop(0, NPTS // CH2, chunk_body, 0)

    def gchunk_body(gi, carry):
        goff = lo + gi * GC
        pltpu.sync_copy(grid_hbm.at[pl.ds(goff, GC)], gb)

        def gvec(j, carry2):
            g = gb[pl.ds(j * L, L)]
            s = slab[pl.ds(gi * GC + j * L, L)]
            touched = s > -0.5
            new = jnp.where(touched,
                            jnp.maximum(g * EMA_D, jnp.maximum(s, 0.0)), g)
            ob[pl.ds(j * L, L)] = new
            return carry2

        lax.fori_loop(0, GC // L, gvec, 0)
        pltpu.sync_copy(ob, out_hbm.at[pl.ds(goff, GC)])
        return carry

    lax.fori_loop(0, SLAB // GC, gchunk_body, 0)


def _thresh_body(x_ref, o_ref):
    o_ref[...] = x_ref[...] > THRE


def _threshold(new_flat):
    x = new_flat.reshape(2048, 1024)
    out = pl.pallas_call(
        _thresh_body,
        out_shape=jax.ShapeDtypeStruct((2048, 1024), jnp.bool_),
        grid=(8,),
        in_specs=[pl.BlockSpec((256, 1024), lambda i: (i, 0))],
        out_specs=pl.BlockSpec((256, 1024), lambda i: (i, 0)),
    )(x)
    return out.reshape(RES, RES, RES)


def kernel(occ_val_grid, pts, val):
    grid_flat = occ_val_grid.reshape(-1)
    flat = _flat_idx_kernel(pts.reshape(-1))
    new_flat = _scatter_ema_kernel(grid_flat, flat, val)
    occ = _threshold(new_flat)
    return new_flat.reshape(RES, RES, RES), occ


# speedup vs baseline: 1.5288x; 1.3660x over previous
import functools

import jax
import jax.numpy as jnp
from jax import lax
from jax.experimental import pallas as pl
from jax.experimental.pallas import tpu as pltpu
from jax.experimental.pallas import tpu_sc as plsc

RES = 128
EMA_D = 0.95
THRE = 0.01
NPTS = 1048576
NCELL = RES * RES * RES
NW = 32
PPT = NPTS // NW
SLAB = NCELL // NW
CH1 = 8192
CH2 = 8192
GC = 8192
L = 16


def _wid():
    return lax.axis_index("s") * 2 + lax.axis_index("c")


@functools.partial(
    pl.kernel,
    out_type=jax.ShapeDtypeStruct((NPTS,), jnp.int32),
    mesh=plsc.VectorSubcoreMesh(core_axis_name="c", subcore_axis_name="s"),
    compiler_params=pltpu.CompilerParams(needs_layout_passes=False),
    scratch_types=[
        pltpu.VMEM((CH1,), jnp.float32),
        pltpu.VMEM((CH1,), jnp.float32),
        pltpu.VMEM((CH1,), jnp.float32),
        pltpu.VMEM((CH1,), jnp.int32),
    ],
)
def _flat_idx_kernel(x_hbm, y_hbm, z_hbm, flat_hbm, xb, yb, zb, flatb):
    w = _wid()
    base = w * PPT

    def chunk_body(ci, carry):
        off = base + ci * CH1
        pltpu.sync_copy(x_hbm.at[pl.ds(off, CH1)], xb)
        pltpu.sync_copy(y_hbm.at[pl.ds(off, CH1)], yb)
        pltpu.sync_copy(z_hbm.at[pl.ds(off, CH1)], zb)

        def vec_body(j, carry2):
            s = pl.ds(j * L, L)

            def quant(t):
                tf = (t * 0.5 + 0.5) * 128.0
                ti = tf.astype(jnp.int32)
                return jnp.clip(ti, 0, RES - 1)

            f = (quant(xb[s]) * RES + quant(yb[s])) * RES + quant(zb[s])
            flatb[s] = f
            return carry2

        lax.fori_loop(0, CH1 // L, vec_body, 0)
        pltpu.sync_copy(flatb, flat_hbm.at[pl.ds(off, CH1)])
        return carry

    lax.fori_loop(0, PPT // CH1, chunk_body, 0)


@functools.partial(
    pl.kernel,
    out_type=jax.ShapeDtypeStruct((NCELL,), jnp.float32),
    mesh=plsc.VectorSubcoreMesh(core_axis_name="c", subcore_axis_name="s"),
    compiler_params=pltpu.CompilerParams(needs_layout_passes=False),
    scratch_types=[
        pltpu.VMEM((SLAB,), jnp.float32),
        pltpu.VMEM((CH2,), jnp.int32),
        pltpu.VMEM((CH2,), jnp.float32),
        pltpu.VMEM((GC,), jnp.float32),
        pltpu.VMEM((GC,), jnp.float32),
    ],
)
def _scatter_ema_kernel(grid_hbm, flat_hbm, val_hbm, out_hbm,
                        slab, flatb, valb, gb, ob):
    w = _wid()
    lo = w * SLAB
    neg1 = jnp.full((L,), -1.0, jnp.float32)

    def init_body(i, carry):
        slab[pl.ds(i * L, L)] = neg1
        return carry

    lax.fori_loop(0, SLAB // L, init_body, 0)

    def chunk_body(ci, carry):
        off = ci * CH2
        pltpu.sync_copy(flat_hbm.at[pl.ds(off, CH2)], flatb)
        pltpu.sync_copy(val_hbm.at[pl.ds(off, CH2)], valb)

        def vec_body(j, carry2):
            f = flatb[pl.ds(j * L, L)]
            v = valb[pl.ds(j * L, L)]
            loc = f - lo
            m = (loc >= 0) & (loc < SLAB)
            locc = jnp.clip(loc, 0, SLAB - 1)
            cnt = jnp.sum(m.astype(jnp.int32))

            @pl.when(cnt > 0)
            def _():
                g = plsc.load_gather(slab, [locc], mask=m)
                plsc.store_scatter(slab, [locc], jnp.maximum(g, v), mask=m)
                g2 = plsc.load_gather(slab, [locc], mask=m)
                need = m & (g2 < v)

                def rcond(nd):
                    return jnp.sum(nd.astype(jnp.int32)) > 0

                def rbody(nd):
                    gg = plsc.load_gather(slab, [locc], mask=nd)
                    plsc.store_scatter(slab, [locc], jnp.maximum(gg, v),
                                       mask=nd)
                    gg2 = plsc.load_gather(slab, [locc], mask=nd)
                    return nd & (gg2 < v)

                lax.while_loop(rcond, rbody, need)

            return carry2

        lax.fori_loop(0, CH2 // L, vec_body, 0)
        return carry

    lax.fori_loop(0, NPTS // CH2, chunk_body, 0)

    def gchunk_body(gi, carry):
        goff = lo + gi * GC
        pltpu.sync_copy(grid_hbm.at[pl.ds(goff, GC)], gb)

        def gvec(j, carry2):
            g = gb[pl.ds(j * L, L)]
            s = slab[pl.ds(gi * GC + j * L, L)]
            touched = s > -0.5
            new = jnp.where(touched,
                            jnp.maximum(g * EMA_D, jnp.maximum(s, 0.0)), g)
            ob[pl.ds(j * L, L)] = new
            return carry2

        lax.fori_loop(0, GC // L, gvec, 0)
        pltpu.sync_copy(ob, out_hbm.at[pl.ds(goff, GC)])
        return carry

    lax.fori_loop(0, SLAB // GC, gchunk_body, 0)


def _thresh_body(x_ref, o_ref):
    o_ref[...] = x_ref[...] > THRE


def _threshold(new_grid):
    return pl.pallas_call(
        _thresh_body,
        out_shape=jax.ShapeDtypeStruct((RES, RES, RES), jnp.bool_),
        grid=(4,),
        in_specs=[pl.BlockSpec((32, RES, RES), lambda i: (i, 0, 0))],
        out_specs=pl.BlockSpec((32, RES, RES), lambda i: (i, 0, 0)),
    )(new_grid)


def kernel(occ_val_grid, pts, val):
    grid_flat = occ_val_grid.reshape(-1)
    pts_t = pts.T
    flat = _flat_idx_kernel(pts_t[0], pts_t[1], pts_t[2])
    new_flat = _scatter_ema_kernel(grid_flat, flat, val)
    new_grid = new_flat.reshape(RES, RES, RES)
    occ = _threshold(new_grid)
    return new_grid, occ


# speedup vs baseline: 18.0545x; 11.8099x over previous
import functools

import jax
import jax.numpy as jnp
from jax import lax
from jax.experimental import pallas as pl
from jax.experimental.pallas import tpu as pltpu
from jax.experimental.pallas import tpu_sc as plsc

RES = 128
EMA_D = 0.95
THRE = 0.01
NPTS = 1048576
NCELL = RES * RES * RES
NW = 32
PPT = NPTS // NW
SLAB = NCELL // NW
L = 16
CH1 = 4096
CB = 2048
GC = 8192
ASLOT = PPT + 8 * NW
ASZ = NW * ASLOT + CB
MSZ = NW * 64

_SC_MESH = dict(
    mesh=plsc.VectorSubcoreMesh(core_axis_name="c", subcore_axis_name="s"),
    compiler_params=pltpu.CompilerParams(needs_layout_passes=False),
)


def _wid():
    return lax.axis_index("s") * 2 + lax.axis_index("c")


def _quant(t):
    tf = (t * 0.5 + 0.5) * 128.0
    ti = tf.astype(jnp.int32)
    return jnp.clip(ti, 0, RES - 1)


@functools.partial(
    pl.kernel,
    out_type=(
        jax.ShapeDtypeStruct((ASZ,), jnp.int32),
        jax.ShapeDtypeStruct((ASZ,), jnp.float32),
        jax.ShapeDtypeStruct((MSZ,), jnp.int32),
    ),
    scratch_types=[
        pltpu.VMEM((PPT,), jnp.int32),
        pltpu.VMEM((ASLOT,), jnp.int32),
        pltpu.VMEM((ASLOT,), jnp.float32),
        pltpu.VMEM((CH1,), jnp.float32),
        pltpu.VMEM((CH1,), jnp.float32),
        pltpu.VMEM((CH1,), jnp.float32),
        pltpu.VMEM((CH1,), jnp.float32),
        pltpu.VMEM((512,), jnp.int32),
        pltpu.VMEM((512,), jnp.int32),
        pltpu.VMEM((64,), jnp.int32),
    ],
    **_SC_MESH,
)
def _route_kernel(x_hbm, y_hbm, z_hbm, val_hbm, a_hbm, b_hbm, m_hbm,
                  flat_all, sf, sv, xb, yb, zb, valb, hist2, cur2, meta):
    w = _wid()
    base_pt = w * PPT
    lane = lax.iota(jnp.int32, L)
    zero16 = jnp.zeros((L,), jnp.int32)
    one16 = jnp.ones((L,), jnp.int32)

    def zinit(i, c):
        hist2[pl.ds(i * L, L)] = zero16
        return c

    lax.fori_loop(0, 512 // L, zinit, 0)

    def p1chunk(ci, c):
        off = ci * CH1
        pltpu.sync_copy(x_hbm.at[pl.ds(base_pt + off, CH1)], xb)
        pltpu.sync_copy(y_hbm.at[pl.ds(base_pt + off, CH1)], yb)
        pltpu.sync_copy(z_hbm.at[pl.ds(base_pt + off, CH1)], zb)

        def vb(j, c2):
            s = pl.ds(j * L, L)
            f = (_quant(xb[s]) * RES + _quant(yb[s])) * RES + _quant(zb[s])
            flat_all[pl.ds(off + j * L, L)] = f
            ha = lane * 32 + (f >> 16)
            plsc.addupdate_scatter(hist2, [ha], one16)
            return c2

        lax.fori_loop(0, CH1 // L, vb, 0)
        return c

    lax.fori_loop(0, PPT // CH1, p1chunk, 0)

    def acc(l, carry):
        t0, t1 = carry
        return (t0 + hist2[pl.ds(l * 32, L)],
                t1 + hist2[pl.ds(l * 32 + L, L)])

    t0, t1 = lax.fori_loop(0, L, acc, (zero16, zero16))
    p0 = ((t0 + 7) >> 3) << 3
    p1 = ((t1 + 7) >> 3) << 3
    c0 = jnp.cumsum(p0)
    base0 = c0 - p0
    s0 = jnp.sum(p0)
    c1 = jnp.cumsum(p1)
    base1 = c1 - p1 + s0
    meta[pl.ds(0, L)] = base0
    meta[pl.ds(L, L)] = base1
    meta[pl.ds(2 * L, L)] = t0
    meta[pl.ds(3 * L, L)] = t1
    pltpu.sync_copy(meta, m_hbm.at[pl.ds(w * 64, 64)])

    def curloop(l, carry):
        a0, a1 = carry
        cur2[pl.ds(l * 32, L)] = a0
        cur2[pl.ds(l * 32 + L, L)] = a1
        return (a0 + hist2[pl.ds(l * 32, L)],
                a1 + hist2[pl.ds(l * 32 + L, L)])

    lax.fori_loop(0, L, curloop, (base0, base1))

    def p2chunk(ci, c):
        off = ci * CH1
        pltpu.sync_copy(val_hbm.at[pl.ds(base_pt + off, CH1)], valb)

        def vb(j, c2):
            f = flat_all[pl.ds(off + j * L, L)]
            v = valb[pl.ds(j * L, L)]
            ha = lane * 32 + (f >> 16)
            pos = plsc.load_gather(cur2, [ha])
            plsc.store_scatter(cur2, [ha], pos + 1)
            plsc.store_scatter(sf, [pos], f)
            plsc.store_scatter(sv, [pos], v)
            return c2

        lax.fori_loop(0, CH1 // L, vb, 0)
        return c

    lax.fori_loop(0, PPT // CH1, p2chunk, 0)

    pltpu.sync_copy(sf, a_hbm.at[pl.ds(w * ASLOT, ASLOT)])
    pltpu.sync_copy(sv, b_hbm.at[pl.ds(w * ASLOT, ASLOT)])


@functools.partial(
    pl.kernel,
    out_type=jax.ShapeDtypeStruct((NCELL,), jnp.float32),
    scratch_types=[
        pltpu.VMEM((SLAB,), jnp.float32),
        pltpu.VMEM((CB,), jnp.int32),
        pltpu.VMEM((CB,), jnp.float32),
        pltpu.VMEM((MSZ,), jnp.int32),
        pltpu.VMEM((GC,), jnp.float32),
        pltpu.VMEM((GC,), jnp.float32),
    ],
    **_SC_MESH,
)
def _merge_ema_kernel(grid_hbm, a_hbm, b_hbm, m_hbm, out_hbm,
                      slab, fb, vb, mb, gb, ob):
    w = _wid()
    lo = w * SLAB
    lane = lax.iota(jnp.int32, L)
    neg1 = jnp.full((L,), -1.0, jnp.float32)

    pltpu.sync_copy(m_hbm, mb)

    def init_body(i, c):
        slab[pl.ds(i * L, L)] = neg1
        return c

    lax.fori_loop(0, SLAB // L, init_body, 0)

    wv = w % L
    half = (w // L) * L
    sel = lane == wv

    def src_loop(t, c):
        bvec = mb[pl.ds(pl.multiple_of(t * 64 + half, 8), L)]
        cvec = mb[pl.ds(pl.multiple_of(t * 64 + 2 * L + half, 8), L)]
        sbase = pl.multiple_of(jnp.sum(jnp.where(sel, bvec, 0)), 8)
        scnt = jnp.sum(jnp.where(sel, cvec, 0))
        off = pl.multiple_of(t * ASLOT + sbase, 8)
        nch = (scnt + CB - 1) // CB

        def chunk(k, c2):
            pltpu.sync_copy(a_hbm.at[pl.ds(off + k * CB, CB)], fb)
            pltpu.sync_copy(b_hbm.at[pl.ds(off + k * CB, CB)], vb)
            rem0 = scnt - k * CB
            nvec = (jnp.minimum(rem0, CB) + L - 1) // L

            def vloop(j, c3):
                msk = (j * L + lane) < rem0
                f = fb[pl.ds(j * L, L)]
                v = vb[pl.ds(j * L, L)]
                loc = jnp.clip(f - lo, 0, SLAB - 1)
                g = plsc.load_gather(slab, [loc], mask=msk)
                plsc.store_scatter(slab, [loc], jnp.maximum(g, v), mask=msk)
                g2 = plsc.load_gather(slab, [loc], mask=msk)
                need = msk & (g2 < v)

                def rcond(nd):
                    return jnp.sum(nd.astype(jnp.int32)) > 0

                def rbody(nd):
                    gg = plsc.load_gather(slab, [loc], mask=nd)
                    plsc.store_scatter(slab, [loc], jnp.maximum(gg, v),
                                       mask=nd)
                    gg2 = plsc.load_gather(slab, [loc], mask=nd)
                    return nd & (gg2 < v)

                lax.while_loop(rcond, rbody, need)
                return c3

            lax.fori_loop(0, nvec, vloop, 0)
            return c2

        lax.fori_loop(0, nch, chunk, 0)
        return c

    lax.fori_loop(0, NW, src_loop, 0)

    def gchunk_body(gi, c):
        goff = lo + gi * GC
        pltpu.sync_copy(grid_hbm.at[pl.ds(goff, GC)], gb)

        def gvec(j, c2):
            g = gb[pl.ds(j * L, L)]
            s = slab[pl.ds(gi * GC + j * L, L)]
            touched = s > -0.5
            new = jnp.where(touched,
                            jnp.maximum(g * EMA_D, jnp.maximum(s, 0.0)), g)
            ob[pl.ds(j * L, L)] = new
            return c2

        lax.fori_loop(0, GC // L, gvec, 0)
        pltpu.sync_copy(ob, out_hbm.at[pl.ds(goff, GC)])
        return c

    lax.fori_loop(0, SLAB // GC, gchunk_body, 0)


def _thresh_body(x_ref, o_ref):
    o_ref[...] = x_ref[...] > THRE


def _threshold(new_grid):
    return pl.pallas_call(
        _thresh_body,
        out_shape=jax.ShapeDtypeStruct((RES, RES, RES), jnp.bool_),
        grid=(4,),
        in_specs=[pl.BlockSpec((32, RES, RES), lambda i: (i, 0, 0))],
        out_specs=pl.BlockSpec((32, RES, RES), lambda i: (i, 0, 0)),
    )(new_grid)


def kernel(occ_val_grid, pts, val):
    grid_flat = occ_val_grid.reshape(-1)
    pts_t = pts.T
    a, b, m = _route_kernel(pts_t[0], pts_t[1], pts_t[2], val)
    new_flat = _merge_ema_kernel(grid_flat, a, b, m)
    new_grid = new_flat.reshape(RES, RES, RES)
    occ = _threshold(new_grid)
    return new_grid, occ


# speedup vs baseline: 22.4126x; 1.2414x over previous
import functools

import jax
import jax.numpy as jnp
from jax import lax
from jax.experimental import pallas as pl
from jax.experimental.pallas import tpu as pltpu
from jax.experimental.pallas import tpu_sc as plsc

RES = 128
EMA_D = 0.95
THRE = 0.01
NPTS = 1048576
NCELL = RES * RES * RES
NW = 32
PPT = NPTS // NW
SLAB = NCELL // NW
L = 16
CH1 = 2048
CB = 2048
GC = 8192
ASLOT = PPT + 8 * NW
ASZ = NW * ASLOT + CB
MSZ = NW * 64

_SC_MESH = dict(
    mesh=plsc.VectorSubcoreMesh(core_axis_name="c", subcore_axis_name="s"),
    compiler_params=pltpu.CompilerParams(needs_layout_passes=False),
)


def _wid():
    return lax.axis_index("s") * 2 + lax.axis_index("c")


def _quant(t):
    tf = (t * 0.5 + 0.5) * 128.0
    ti = tf.astype(jnp.int32)
    return jnp.clip(ti, 0, RES - 1)


@functools.partial(
    pl.kernel,
    out_type=(
        jax.ShapeDtypeStruct((ASZ,), jnp.int32),
        jax.ShapeDtypeStruct((ASZ,), jnp.float32),
        jax.ShapeDtypeStruct((MSZ,), jnp.int32),
    ),
    scratch_types=[
        pltpu.VMEM((PPT,), jnp.int32),
        pltpu.VMEM((ASLOT,), jnp.int32),
        pltpu.VMEM((ASLOT,), jnp.float32),
        pltpu.VMEM((CH1,), jnp.float32),
        pltpu.VMEM((CH1,), jnp.float32),
        pltpu.VMEM((CH1,), jnp.float32),
        pltpu.VMEM((CH1,), jnp.float32),
        pltpu.VMEM((CH1,), jnp.float32),
        pltpu.VMEM((CH1,), jnp.float32),
        pltpu.VMEM((CH1,), jnp.float32),
        pltpu.VMEM((CH1,), jnp.float32),
        pltpu.VMEM((512,), jnp.int32),
        pltpu.VMEM((512,), jnp.int32),
        pltpu.VMEM((64,), jnp.int32),
        pltpu.SemaphoreType.DMA,
        pltpu.SemaphoreType.DMA,
        pltpu.SemaphoreType.DMA,
        pltpu.SemaphoreType.DMA,
        pltpu.SemaphoreType.DMA,
    ],
    **_SC_MESH,
)
def _route_kernel(x_hbm, y_hbm, z_hbm, val_hbm, a_hbm, b_hbm, m_hbm,
                  flat_all, sf, sv, xb0, yb0, zb0, xb1, yb1, zb1,
                  valb0, valb1, hist2, cur2, meta,
                  sem0, sem1, vsem0, vsem1, osem):
    w = _wid()
    base_pt = w * PPT
    lane = lax.iota(jnp.int32, L)
    zero16 = jnp.zeros((L,), jnp.int32)
    one16 = jnp.ones((L,), jnp.int32)
    NCH = PPT // CH1

    def zinit(i, c):
        hist2[pl.ds(i * L, L)] = zero16
        return c

    lax.fori_loop(0, 512 // L, zinit, 0, unroll=4)

    bufs = ((xb0, yb0, zb0, sem0), (xb1, yb1, zb1, sem1))

    def start1(ci):
        xb, yb, zb, sem = bufs[ci % 2]
        s = pl.ds(base_pt + ci * CH1, CH1)
        return (pltpu.async_copy(x_hbm.at[s], xb, sem),
                pltpu.async_copy(y_hbm.at[s], yb, sem),
                pltpu.async_copy(z_hbm.at[s], zb, sem))

    pend = start1(0)
    for ci in range(NCH):
        nxt = start1(ci + 1) if ci + 1 < NCH else None
        for h in pend:
            h.wait()
        xb, yb, zb, _ = bufs[ci % 2]
        off = ci * CH1

        def vb1(j, c2):
            s = pl.ds(j * L, L)
            f = (_quant(xb[s]) * RES + _quant(yb[s])) * RES + _quant(zb[s])
            flat_all[pl.ds(off + j * L, L)] = f
            ha = lane * 32 + (f >> 16)
            plsc.addupdate_scatter(hist2, [ha], one16)
            return c2

        lax.fori_loop(0, CH1 // L, vb1, 0, unroll=4)
        pend = nxt

    def acc(l, carry):
        t0, t1 = carry
        return (t0 + hist2[pl.ds(l * 32, L)],
                t1 + hist2[pl.ds(l * 32 + L, L)])

    t0, t1 = lax.fori_loop(0, L, acc, (zero16, zero16), unroll=4)
    p0 = ((t0 + 7) >> 3) << 3
    p1 = ((t1 + 7) >> 3) << 3
    c0 = jnp.cumsum(p0)
    base0 = c0 - p0
    s0 = jnp.sum(p0)
    c1 = jnp.cumsum(p1)
    base1 = c1 - p1 + s0
    meta[pl.ds(0, L)] = base0
    meta[pl.ds(L, L)] = base1
    meta[pl.ds(2 * L, L)] = t0
    meta[pl.ds(3 * L, L)] = t1
    mh = pltpu.async_copy(meta, m_hbm.at[pl.ds(w * 64, 64)], osem)

    def curloop(l, carry):
        a0, a1 = carry
        cur2[pl.ds(l * 32, L)] = a0
        cur2[pl.ds(l * 32 + L, L)] = a1
        return (a0 + hist2[pl.ds(l * 32, L)],
                a1 + hist2[pl.ds(l * 32 + L, L)])

    lax.fori_loop(0, L, curloop, (base0, base1), unroll=4)

    vbufs = ((valb0, vsem0), (valb1, vsem1))

    def start2(ci):
        vbuf, sem = vbufs[ci % 2]
        s = pl.ds(base_pt + ci * CH1, CH1)
        return pltpu.async_copy(val_hbm.at[s], vbuf, sem)

    vpend = start2(0)
    for ci in range(NCH):
        vnxt = start2(ci + 1) if ci + 1 < NCH else None
        vpend.wait()
        valb, _ = vbufs[ci % 2]
        off = ci * CH1

        def vb2(j, c2):
            f = flat_all[pl.ds(off + j * L, L)]
            v = valb[pl.ds(j * L, L)]
            ha = lane * 32 + (f >> 16)
            pos = plsc.load_gather(cur2, [ha])
            plsc.store_scatter(cur2, [ha], pos + 1)
            plsc.store_scatter(sf, [pos], f)
            plsc.store_scatter(sv, [pos], v)
            return c2

        lax.fori_loop(0, CH1 // L, vb2, 0, unroll=4)
        vpend = vnxt

    ah = pltpu.async_copy(sf, a_hbm.at[pl.ds(w * ASLOT, ASLOT)], osem)
    bh = pltpu.async_copy(sv, b_hbm.at[pl.ds(w * ASLOT, ASLOT)], osem)
    mh.wait()
    ah.wait()
    bh.wait()


@functools.partial(
    pl.kernel,
    out_type=jax.ShapeDtypeStruct((NCELL,), jnp.float32),
    scratch_types=[
        pltpu.VMEM((SLAB,), jnp.float32),
        pltpu.VMEM((CB,), jnp.int32),
        pltpu.VMEM((CB,), jnp.float32),
        pltpu.VMEM((MSZ,), jnp.int32),
        pltpu.VMEM((GC,), jnp.float32),
        pltpu.VMEM((GC,), jnp.float32),
        pltpu.VMEM((GC,), jnp.float32),
        pltpu.VMEM((GC,), jnp.float32),
        pltpu.SemaphoreType.DMA,
        pltpu.SemaphoreType.DMA,
        pltpu.SemaphoreType.DMA,
        pltpu.SemaphoreType.DMA,
        pltpu.SemaphoreType.DMA,
        pltpu.SemaphoreType.DMA,
        pltpu.SemaphoreType.DMA,
    ],
    **_SC_MESH,
)
def _merge_ema_kernel(grid_hbm, a_hbm, b_hbm, m_hbm, out_hbm,
                      slab, fb, vb, mb, gb0, gb1, ob0, ob1,
                      fsem, vsem, gin0, gin1, gout0, gout1, msem):
    w = _wid()
    lo = w * SLAB
    lane = lax.iota(jnp.int32, L)
    neg1 = jnp.full((L,), -1.0, jnp.float32)

    mbh = pltpu.async_copy(m_hbm, mb, msem)

    def init_body(i, c):
        slab[pl.ds(i * L, L)] = neg1
        return c

    lax.fori_loop(0, SLAB // L, init_body, 0, unroll=8)
    mbh.wait()

    wv = w % L
    half = (w // L) * L
    sel = lane == wv

    def src_loop(t, c):
        bvec = mb[pl.ds(pl.multiple_of(t * 64 + half, 8), L)]
        cvec = mb[pl.ds(pl.multiple_of(t * 64 + 2 * L + half, 8), L)]
        sbase = pl.multiple_of(jnp.sum(jnp.where(sel, bvec, 0)), 8)
        scnt = jnp.sum(jnp.where(sel, cvec, 0))
        off = pl.multiple_of(t * ASLOT + sbase, 8)
        nch = (scnt + CB - 1) // CB

        def chunk(k, c2):
            s = pl.ds(off + k * CB, CB)
            fh = pltpu.async_copy(a_hbm.at[s], fb, fsem)
            vh = pltpu.async_copy(b_hbm.at[s], vb, vsem)
            fh.wait()
            vh.wait()
            rem0 = scnt - k * CB
            nvec = (jnp.minimum(rem0, CB) + L - 1) // L

            def vloop(j, c3):
                msk = (j * L + lane) < rem0
                f = fb[pl.ds(j * L, L)]
                v = vb[pl.ds(j * L, L)]
                loc = jnp.clip(f - lo, 0, SLAB - 1)
                g = plsc.load_gather(slab, [loc], mask=msk)
                plsc.store_scatter(slab, [loc], jnp.maximum(g, v), mask=msk)
                g2 = plsc.load_gather(slab, [loc], mask=msk)
                need = msk & (g2 < v)

                def rcond(nd):
                    return jnp.sum(nd.astype(jnp.int32)) > 0

                def rbody(nd):
                    gg = plsc.load_gather(slab, [loc], mask=nd)
                    plsc.store_scatter(slab, [loc], jnp.maximum(gg, v),
                                       mask=nd)
                    gg2 = plsc.load_gather(slab, [loc], mask=nd)
                    return nd & (gg2 < v)

                lax.while_loop(rcond, rbody, need)
                return c3

            lax.fori_loop(0, nvec, vloop, 0)
            return c2

        lax.fori_loop(0, nch, chunk, 0)
        return c

    lax.fori_loop(0, NW, src_loop, 0)

    NG = SLAB // GC
    gbufs = ((gb0, ob0, gin0, gout0), (gb1, ob1, gin1, gout1))

    def start_g(gi):
        gb, _, gin, _ = gbufs[gi % 2]
        return pltpu.async_copy(grid_hbm.at[pl.ds(lo + gi * GC, GC)], gb, gin)

    gpend = start_g(0)
    opend = [None, None]
    for gi in range(NG):
        gnxt = start_g(gi + 1) if gi + 1 < NG else None
        gpend.wait()
        gb, ob, _, gout = gbufs[gi % 2]
        if opend[gi % 2] is not None:
            opend[gi % 2].wait()

        def gvec(j, c2, gb=gb, ob=ob, gi=gi):
            g = gb[pl.ds(j * L, L)]
            s = slab[pl.ds(gi * GC + j * L, L)]
            touched = s > -0.5
            new = jnp.where(touched,
                            jnp.maximum(g * EMA_D, jnp.maximum(s, 0.0)), g)
            ob[pl.ds(j * L, L)] = new
            return c2

        lax.fori_loop(0, GC // L, gvec, 0, unroll=4)
        opend[gi % 2] = pltpu.async_copy(
            ob, out_hbm.at[pl.ds(lo + gi * GC, GC)], gout)
        gpend = gnxt
    for h in opend:
        if h is not None:
            h.wait()


def _thresh_body(x_ref, o_ref):
    o_ref[...] = x_ref[...] > THRE


def _threshold(new_grid):
    return pl.pallas_call(
        _thresh_body,
        out_shape=jax.ShapeDtypeStruct((RES, RES, RES), jnp.bool_),
        grid=(4,),
        in_specs=[pl.BlockSpec((32, RES, RES), lambda i: (i, 0, 0))],
        out_specs=pl.BlockSpec((32, RES, RES), lambda i: (i, 0, 0)),
    )(new_grid)


def kernel(occ_val_grid, pts, val):
    grid_flat = occ_val_grid.reshape(-1)
    pts_t = pts.T
    a, b, m = _route_kernel(pts_t[0], pts_t[1], pts_t[2], val)
    new_flat = _merge_ema_kernel(grid_flat, a, b, m)
    new_grid = new_flat.reshape(RES, RES, RES)
    occ = _threshold(new_grid)
    return new_grid, occ


# speedup vs baseline: 25.9126x; 1.1562x over previous
import functools

import jax
import jax.numpy as jnp
from jax import lax
from jax.experimental import pallas as pl
from jax.experimental.pallas import tpu as pltpu
from jax.experimental.pallas import tpu_sc as plsc

RES = 128
EMA_D = 0.95
THRE = 0.01
NPTS = 1048576
NCELL = RES * RES * RES
NW = 32
PPT = NPTS // NW
SLAB = NCELL // NW
L = 16
CH1 = 2048
CB = 2048
GC = 8192
ASLOT = PPT + 8 * NW
ASZ = NW * ASLOT + CB
MSZ = NW * 64

_SC_MESH = dict(
    mesh=plsc.VectorSubcoreMesh(core_axis_name="c", subcore_axis_name="s"),
    compiler_params=pltpu.CompilerParams(needs_layout_passes=False),
)


def _wid():
    return lax.axis_index("s") * 2 + lax.axis_index("c")


def _quant(t):
    tf = (t * 0.5 + 0.5) * 128.0
    ti = tf.astype(jnp.int32)
    return jnp.clip(ti, 0, RES - 1)


@functools.partial(
    pl.kernel,
    out_type=(
        jax.ShapeDtypeStruct((ASZ,), jnp.int32),
        jax.ShapeDtypeStruct((ASZ,), jnp.float32),
        jax.ShapeDtypeStruct((MSZ,), jnp.int32),
    ),
    scratch_types=[
        pltpu.VMEM((PPT,), jnp.int32),
        pltpu.VMEM((ASLOT,), jnp.int32),
        pltpu.VMEM((ASLOT,), jnp.float32),
        pltpu.VMEM((CH1,), jnp.float32),
        pltpu.VMEM((CH1,), jnp.float32),
        pltpu.VMEM((CH1,), jnp.float32),
        pltpu.VMEM((CH1,), jnp.float32),
        pltpu.VMEM((CH1,), jnp.float32),
        pltpu.VMEM((CH1,), jnp.float32),
        pltpu.VMEM((CH1,), jnp.float32),
        pltpu.VMEM((CH1,), jnp.float32),
        pltpu.VMEM((512,), jnp.int32),
        pltpu.VMEM((512,), jnp.int32),
        pltpu.VMEM((64,), jnp.int32),
        pltpu.SemaphoreType.DMA,
        pltpu.SemaphoreType.DMA,
        pltpu.SemaphoreType.DMA,
        pltpu.SemaphoreType.DMA,
        pltpu.SemaphoreType.DMA,
    ],
    **_SC_MESH,
)
def _route_kernel(x_hbm, y_hbm, z_hbm, val_hbm, a_hbm, b_hbm, m_hbm,
                  flat_all, sf, sv, xb0, yb0, zb0, xb1, yb1, zb1,
                  valb0, valb1, hist2, cur2, meta,
                  sem0, sem1, vsem0, vsem1, osem):
    w = _wid()
    base_pt = w * PPT
    lane = lax.iota(jnp.int32, L)
    zero16 = jnp.zeros((L,), jnp.int32)
    one16 = jnp.ones((L,), jnp.int32)
    NCH = PPT // CH1

    def zinit(i, c):
        hist2[pl.ds(i * L, L)] = zero16
        return c

    lax.fori_loop(0, 512 // L, zinit, 0, unroll=4)

    bufs = ((xb0, yb0, zb0, sem0), (xb1, yb1, zb1, sem1))

    def start1(ci):
        xb, yb, zb, sem = bufs[ci % 2]
        s = pl.ds(base_pt + ci * CH1, CH1)
        return (pltpu.async_copy(x_hbm.at[s], xb, sem),
                pltpu.async_copy(y_hbm.at[s], yb, sem),
                pltpu.async_copy(z_hbm.at[s], zb, sem))

    pend = start1(0)
    for ci in range(NCH):
        nxt = start1(ci + 1) if ci + 1 < NCH else None
        for h in pend:
            h.wait()
        xb, yb, zb, _ = bufs[ci % 2]
        off = ci * CH1

        def vb1(j, c2):
            s = pl.ds(j * L, L)
            f = (_quant(xb[s]) * RES + _quant(yb[s])) * RES + _quant(zb[s])
            flat_all[pl.ds(off + j * L, L)] = f
            ha = lane * 32 + (f >> 16)
            plsc.addupdate_scatter(hist2, [ha], one16)
            return c2

        lax.fori_loop(0, CH1 // L, vb1, 0, unroll=4)
        pend = nxt

    def acc(l, carry):
        t0, t1 = carry
        return (t0 + hist2[pl.ds(l * 32, L)],
                t1 + hist2[pl.ds(l * 32 + L, L)])

    t0, t1 = lax.fori_loop(0, L, acc, (zero16, zero16), unroll=4)
    p0 = ((t0 + 7) >> 3) << 3
    p1 = ((t1 + 7) >> 3) << 3
    c0 = jnp.cumsum(p0)
    base0 = c0 - p0
    s0 = jnp.sum(p0)
    c1 = jnp.cumsum(p1)
    base1 = c1 - p1 + s0
    meta[pl.ds(0, L)] = base0
    meta[pl.ds(L, L)] = base1
    meta[pl.ds(2 * L, L)] = t0
    meta[pl.ds(3 * L, L)] = t1
    mh = pltpu.async_copy(meta, m_hbm.at[pl.ds(w * 64, 64)], osem)

    def curloop(l, carry):
        a0, a1 = carry
        cur2[pl.ds(l * 32, L)] = a0
        cur2[pl.ds(l * 32 + L, L)] = a1
        return (a0 + hist2[pl.ds(l * 32, L)],
                a1 + hist2[pl.ds(l * 32 + L, L)])

    lax.fori_loop(0, L, curloop, (base0, base1), unroll=4)

    vbufs = ((valb0, vsem0), (valb1, vsem1))

    def start2(ci):
        vbuf, sem = vbufs[ci % 2]
        s = pl.ds(base_pt + ci * CH1, CH1)
        return pltpu.async_copy(val_hbm.at[s], vbuf, sem)

    vpend = start2(0)
    for ci in range(NCH):
        vnxt = start2(ci + 1) if ci + 1 < NCH else None
        vpend.wait()
        valb, _ = vbufs[ci % 2]
        off = ci * CH1

        def vb2(j, c2):
            f = flat_all[pl.ds(off + j * L, L)]
            v = valb[pl.ds(j * L, L)]
            ha = lane * 32 + (f >> 16)
            pos = plsc.load_gather(cur2, [ha])
            plsc.store_scatter(cur2, [ha], pos + 1)
            plsc.store_scatter(sf, [pos], f)
            plsc.store_scatter(sv, [pos], v)
            return c2

        lax.fori_loop(0, CH1 // L, vb2, 0, unroll=4)
        vpend = vnxt

    ah = pltpu.async_copy(sf, a_hbm.at[pl.ds(w * ASLOT, ASLOT)], osem)
    bh = pltpu.async_copy(sv, b_hbm.at[pl.ds(w * ASLOT, ASLOT)], osem)
    mh.wait()
    ah.wait()
    bh.wait()


@functools.partial(
    pl.kernel,
    out_type=jax.ShapeDtypeStruct((NCELL,), jnp.float32),
    scratch_types=[
        pltpu.VMEM((SLAB,), jnp.float32),
        pltpu.VMEM((CB,), jnp.int32),
        pltpu.VMEM((CB,), jnp.float32),
        pltpu.VMEM((MSZ,), jnp.int32),
        pltpu.VMEM((GC,), jnp.float32),
        pltpu.VMEM((GC,), jnp.float32),
        pltpu.VMEM((GC,), jnp.float32),
        pltpu.VMEM((GC,), jnp.float32),
        pltpu.SemaphoreType.DMA,
        pltpu.SemaphoreType.DMA,
        pltpu.SemaphoreType.DMA,
        pltpu.SemaphoreType.DMA,
        pltpu.SemaphoreType.DMA,
        pltpu.SemaphoreType.DMA,
        pltpu.SemaphoreType.DMA,
    ],
    **_SC_MESH,
)
def _merge_ema_kernel(grid_hbm, a_hbm, b_hbm, m_hbm, out_hbm,
                      slab, fb, vb, mb, gb0, gb1, ob0, ob1,
                      fsem, vsem, gin0, gin1, gout0, gout1, msem):
    w = _wid()
    lo = w * SLAB
    lane = lax.iota(jnp.int32, L)
    neg1 = jnp.full((L,), -1.0, jnp.float32)

    mbh = pltpu.async_copy(m_hbm, mb, msem)

    def init_body(i, c):
        slab[pl.ds(i * L, L)] = neg1
        return c

    lax.fori_loop(0, SLAB // L, init_body, 0, unroll=8)
    mbh.wait()

    wv = w % L
    half = (w // L) * L
    sel = lane == wv

    def src_loop(t, c):
        bvec = mb[pl.ds(pl.multiple_of(t * 64 + half, 8), L)]
        cvec = mb[pl.ds(pl.multiple_of(t * 64 + 2 * L + half, 8), L)]
        sbase = pl.multiple_of(jnp.sum(jnp.where(sel, bvec, 0)), 8)
        scnt = jnp.sum(jnp.where(sel, cvec, 0))
        off = pl.multiple_of(t * ASLOT + sbase, 8)
        nch = (scnt + CB - 1) // CB

        def chunk(k, c2):
            s = pl.ds(off + k * CB, CB)
            fh = pltpu.async_copy(a_hbm.at[s], fb, fsem)
            vh = pltpu.async_copy(b_hbm.at[s], vb, vsem)
            fh.wait()
            vh.wait()
            rem0 = scnt - k * CB
            nvec = (jnp.minimum(rem0, CB) + L - 1) // L
            fvec = jnp.zeros((L,), jnp.bool_)

            def vfast(j, acc):
                msk = (j * L + lane) < rem0
                f = fb[pl.ds(j * L, L)]
                v = vb[pl.ds(j * L, L)]
                loc = jnp.clip(f - lo, 0, SLAB - 1)
                _, lastm = plsc.scan_count(loc, msk)
                g = plsc.load_gather(slab, [loc], mask=msk)
                plsc.store_scatter(slab, [loc], jnp.maximum(g, v), mask=msk)
                return acc | (msk & ~lastm)

            sus = lax.fori_loop(0, nvec, vfast, fvec)

            @pl.when(jnp.sum(sus.astype(jnp.int32)) > 0)
            def _():
                def vslow(j, c3):
                    msk = (j * L + lane) < rem0
                    f = fb[pl.ds(j * L, L)]
                    v = vb[pl.ds(j * L, L)]
                    loc = jnp.clip(f - lo, 0, SLAB - 1)
                    g2 = plsc.load_gather(slab, [loc], mask=msk)
                    need = msk & (g2 < v)

                    def rcond(nd):
                        return jnp.sum(nd.astype(jnp.int32)) > 0

                    def rbody(nd):
                        gg = plsc.load_gather(slab, [loc], mask=nd)
                        plsc.store_scatter(slab, [loc], jnp.maximum(gg, v),
                                           mask=nd)
                        gg2 = plsc.load_gather(slab, [loc], mask=nd)
                        return nd & (gg2 < v)

                    lax.while_loop(rcond, rbody, need)
                    return c3

                lax.fori_loop(0, nvec, vslow, 0)

            return c2

        lax.fori_loop(0, nch, chunk, 0)
        return c

    lax.fori_loop(0, NW, src_loop, 0)

    NG = SLAB // GC
    gbufs = ((gb0, ob0, gin0, gout0), (gb1, ob1, gin1, gout1))

    def start_g(gi):
        gb, _, gin, _ = gbufs[gi % 2]
        return pltpu.async_copy(grid_hbm.at[pl.ds(lo + gi * GC, GC)], gb, gin)

    gpend = start_g(0)
    opend = [None, None]
    for gi in range(NG):
        gnxt = start_g(gi + 1) if gi + 1 < NG else None
        gpend.wait()
        gb, ob, _, gout = gbufs[gi % 2]
        if opend[gi % 2] is not None:
            opend[gi % 2].wait()

        def gvec(j, c2, gb=gb, ob=ob, gi=gi):
            g = gb[pl.ds(j * L, L)]
            s = slab[pl.ds(gi * GC + j * L, L)]
            touched = s > -0.5
            new = jnp.where(touched,
                            jnp.maximum(g * EMA_D, jnp.maximum(s, 0.0)), g)
            ob[pl.ds(j * L, L)] = new
            return c2

        lax.fori_loop(0, GC // L, gvec, 0, unroll=4)
        opend[gi % 2] = pltpu.async_copy(
            ob, out_hbm.at[pl.ds(lo + gi * GC, GC)], gout)
        gpend = gnxt
    for h in opend:
        if h is not None:
            h.wait()


def _thresh_body(x_ref, o_ref):
    o_ref[...] = x_ref[...] > THRE


def _threshold(new_grid):
    return pl.pallas_call(
        _thresh_body,
        out_shape=jax.ShapeDtypeStruct((RES, RES, RES), jnp.bool_),
        grid=(4,),
        in_specs=[pl.BlockSpec((32, RES, RES), lambda i: (i, 0, 0))],
        out_specs=pl.BlockSpec((32, RES, RES), lambda i: (i, 0, 0)),
    )(new_grid)


def kernel(occ_val_grid, pts, val):
    grid_flat = occ_val_grid.reshape(-1)
    pts_t = pts.T
    a, b, m = _route_kernel(pts_t[0], pts_t[1], pts_t[2], val)
    new_flat = _merge_ema_kernel(grid_flat, a, b, m)
    new_grid = new_flat.reshape(RES, RES, RES)
    occ = _threshold(new_grid)
    return new_grid, occ


# speedup vs baseline: 27.7650x; 1.0715x over previous
import functools

import jax
import jax.numpy as jnp
from jax import lax
from jax.experimental import pallas as pl
from jax.experimental.pallas import tpu as pltpu
from jax.experimental.pallas import tpu_sc as plsc

RES = 128
EMA_D = 0.95
THRE = 0.01
NPTS = 1048576
NCELL = RES * RES * RES
NW = 32
PPT = NPTS // NW
SLAB = NCELL // NW
L = 16
CH1 = 2048
CB = 2048
GC = 8192
ASLOT = PPT + 8 * NW
ASZ = NW * ASLOT + CB
MSZ = NW * 64

_SC_MESH = dict(
    mesh=plsc.VectorSubcoreMesh(core_axis_name="c", subcore_axis_name="s"),
    compiler_params=pltpu.CompilerParams(needs_layout_passes=False),
)


def _wid():
    return lax.axis_index("s") * 2 + lax.axis_index("c")


def _quant(t):
    tf = (t * 0.5 + 0.5) * 128.0
    ti = tf.astype(jnp.int32)
    return jnp.clip(ti, 0, RES - 1)


@functools.partial(
    pl.kernel,
    out_type=(
        jax.ShapeDtypeStruct((ASZ,), jnp.int32),
        jax.ShapeDtypeStruct((ASZ,), jnp.float32),
        jax.ShapeDtypeStruct((MSZ,), jnp.int32),
    ),
    scratch_types=[
        pltpu.VMEM((PPT,), jnp.int32),
        pltpu.VMEM((ASLOT,), jnp.int32),
        pltpu.VMEM((ASLOT,), jnp.float32),
        pltpu.VMEM((CH1,), jnp.float32),
        pltpu.VMEM((CH1,), jnp.float32),
        pltpu.VMEM((CH1,), jnp.float32),
        pltpu.VMEM((CH1,), jnp.float32),
        pltpu.VMEM((CH1,), jnp.float32),
        pltpu.VMEM((CH1,), jnp.float32),
        pltpu.VMEM((CH1,), jnp.float32),
        pltpu.VMEM((CH1,), jnp.float32),
        pltpu.VMEM((512,), jnp.int32),
        pltpu.VMEM((512,), jnp.int32),
        pltpu.VMEM((64,), jnp.int32),
        pltpu.SemaphoreType.DMA,
        pltpu.SemaphoreType.DMA,
        pltpu.SemaphoreType.DMA,
        pltpu.SemaphoreType.DMA,
        pltpu.SemaphoreType.DMA,
    ],
    **_SC_MESH,
)
def _route_kernel(x_hbm, y_hbm, z_hbm, val_hbm, a_hbm, b_hbm, m_hbm,
                  flat_all, sf, sv, xb0, yb0, zb0, xb1, yb1, zb1,
                  valb0, valb1, hist2, cur2, meta,
                  sem0, sem1, vsem0, vsem1, osem):
    w = _wid()
    base_pt = w * PPT
    lane = lax.iota(jnp.int32, L)
    zero16 = jnp.zeros((L,), jnp.int32)
    one16 = jnp.ones((L,), jnp.int32)
    NCH = PPT // CH1

    def zinit(i, c):
        hist2[pl.ds(i * L, L)] = zero16
        return c

    lax.fori_loop(0, 512 // L, zinit, 0, unroll=4)

    bufs = ((xb0, yb0, zb0, sem0), (xb1, yb1, zb1, sem1))

    def start1(ci):
        xb, yb, zb, sem = bufs[ci % 2]
        s = pl.ds(base_pt + ci * CH1, CH1)
        return (pltpu.async_copy(x_hbm.at[s], xb, sem),
                pltpu.async_copy(y_hbm.at[s], yb, sem),
                pltpu.async_copy(z_hbm.at[s], zb, sem))

    pend = start1(0)
    for ci in range(NCH):
        nxt = start1(ci + 1) if ci + 1 < NCH else None
        for h in pend:
            h.wait()
        xb, yb, zb, _ = bufs[ci % 2]
        off = ci * CH1

        def vb1(j, c2):
            s = pl.ds(j * L, L)
            f = (_quant(xb[s]) * RES + _quant(yb[s])) * RES + _quant(zb[s])
            flat_all[pl.ds(off + j * L, L)] = f
            ha = lane * 32 + (f >> 16)
            plsc.addupdate_scatter(hist2, [ha], one16)
            return c2

        lax.fori_loop(0, CH1 // L, vb1, 0, unroll=4)
        pend = nxt

    def acc(l, carry):
        t0, t1 = carry
        return (t0 + hist2[pl.ds(l * 32, L)],
                t1 + hist2[pl.ds(l * 32 + L, L)])

    t0, t1 = lax.fori_loop(0, L, acc, (zero16, zero16), unroll=4)
    p0 = ((t0 + 7) >> 3) << 3
    p1 = ((t1 + 7) >> 3) << 3
    c0 = jnp.cumsum(p0)
    base0 = c0 - p0
    s0 = jnp.sum(p0)
    c1 = jnp.cumsum(p1)
    base1 = c1 - p1 + s0
    meta[pl.ds(0, L)] = base0
    meta[pl.ds(L, L)] = base1
    meta[pl.ds(2 * L, L)] = t0
    meta[pl.ds(3 * L, L)] = t1
    mh = pltpu.async_copy(meta, m_hbm.at[pl.ds(w * 64, 64)], osem)

    def curloop(l, carry):
        a0, a1 = carry
        cur2[pl.ds(l * 32, L)] = a0
        cur2[pl.ds(l * 32 + L, L)] = a1
        return (a0 + hist2[pl.ds(l * 32, L)],
                a1 + hist2[pl.ds(l * 32 + L, L)])

    lax.fori_loop(0, L, curloop, (base0, base1), unroll=4)

    vbufs = ((valb0, vsem0), (valb1, vsem1))

    def start2(ci):
        vbuf, sem = vbufs[ci % 2]
        s = pl.ds(base_pt + ci * CH1, CH1)
        return pltpu.async_copy(val_hbm.at[s], vbuf, sem)

    vpend = start2(0)
    for ci in range(NCH):
        vnxt = start2(ci + 1) if ci + 1 < NCH else None
        vpend.wait()
        valb, _ = vbufs[ci % 2]
        off = ci * CH1

        def vb2(j, c2):
            f = flat_all[pl.ds(off + j * L, L)]
            v = valb[pl.ds(j * L, L)]
            ha = lane * 32 + (f >> 16)
            pos = plsc.load_gather(cur2, [ha])
            plsc.store_scatter(cur2, [ha], pos + 1)
            plsc.store_scatter(sf, [pos], f)
            plsc.store_scatter(sv, [pos], v)
            return c2

        lax.fori_loop(0, CH1 // L, vb2, 0, unroll=4)
        vpend = vnxt

    ah = pltpu.async_copy(sf, a_hbm.at[pl.ds(w * ASLOT, ASLOT)], osem)
    bh = pltpu.async_copy(sv, b_hbm.at[pl.ds(w * ASLOT, ASLOT)], osem)
    mh.wait()
    ah.wait()
    bh.wait()


@functools.partial(
    pl.kernel,
    out_type=jax.ShapeDtypeStruct((NCELL,), jnp.float32),
    scratch_types=[
        pltpu.VMEM((SLAB,), jnp.float32),
        pltpu.VMEM((CB,), jnp.int32),
        pltpu.VMEM((CB,), jnp.float32),
        pltpu.VMEM((CB,), jnp.int32),
        pltpu.VMEM((CB,), jnp.float32),
        pltpu.VMEM((MSZ,), jnp.int32),
        pltpu.VMEM((GC,), jnp.float32),
        pltpu.VMEM((GC,), jnp.float32),
        pltpu.VMEM((GC,), jnp.float32),
        pltpu.VMEM((GC,), jnp.float32),
        pltpu.SemaphoreType.DMA,
        pltpu.SemaphoreType.DMA,
        pltpu.SemaphoreType.DMA,
        pltpu.SemaphoreType.DMA,
        pltpu.SemaphoreType.DMA,
        pltpu.SemaphoreType.DMA,
        pltpu.SemaphoreType.DMA,
        pltpu.SemaphoreType.DMA,
        pltpu.SemaphoreType.DMA,
        pltpu.SemaphoreType.DMA,
    ],
    **_SC_MESH,
)
def _merge_ema_kernel(grid_hbm, a_hbm, b_hbm, m_hbm, neg_hbm, out_hbm,
                      slab, fb0, vb0, fb1, vb1, mb, gb0, gb1, ob0, ob1,
                      fsem0, vsem0, fsem1, vsem1,
                      gin0, gin1, gout0, gout1, msem, nsem):
    w = _wid()
    lo = w * SLAB
    lane = lax.iota(jnp.int32, L)

    nh = pltpu.async_copy(neg_hbm, slab, nsem)
    mbh = pltpu.async_copy(m_hbm, mb, msem)
    mbh.wait()

    wv = w % L
    half = (w // L) * L
    sel = lane == wv

    def _src_meta(t):
        bvec = mb[pl.ds(pl.multiple_of(t * 64 + half, 8), L)]
        cvec = mb[pl.ds(pl.multiple_of(t * 64 + 2 * L + half, 8), L)]
        sbase = pl.multiple_of(jnp.sum(jnp.where(sel, bvec, 0)), 8)
        scnt = jnp.sum(jnp.where(sel, cvec, 0))
        off = pl.multiple_of(t * ASLOT + sbase, 8)
        return off, scnt

    bufs = ((fb0, vb0, fsem0, vsem0), (fb1, vb1, fsem1, vsem1))

    def _issue(t, off):
        fbx, vbx, fs, vs = bufs[t % 2]
        s = pl.ds(off, CB)
        return (pltpu.async_copy(a_hbm.at[s], fbx, fs),
                pltpu.async_copy(b_hbm.at[s], vbx, vs))

    def _process_chunk(fbx, vbx, rem0):
        nvec = (jnp.minimum(rem0, CB) + L - 1) // L
        fvec = jnp.zeros((L,), jnp.bool_)

        def vfast(j, acc):
            msk = (j * L + lane) < rem0
            f = fbx[pl.ds(j * L, L)]
            v = vbx[pl.ds(j * L, L)]
            loc = jnp.clip(f - lo, 0, SLAB - 1)
            _, lastm = plsc.scan_count(loc, msk)
            g = plsc.load_gather(slab, [loc], mask=msk)
            plsc.store_scatter(slab, [loc], jnp.maximum(g, v), mask=msk)
            return acc | (msk & ~lastm)

        sus = lax.fori_loop(0, nvec, vfast, fvec)

        @pl.when(jnp.sum(sus.astype(jnp.int32)) > 0)
        def _():
            def vslow(j, c3):
                msk = (j * L + lane) < rem0
                f = fbx[pl.ds(j * L, L)]
                v = vbx[pl.ds(j * L, L)]
                loc = jnp.clip(f - lo, 0, SLAB - 1)
                g2 = plsc.load_gather(slab, [loc], mask=msk)
                need = msk & (g2 < v)

                def rcond(nd):
                    return jnp.sum(nd.astype(jnp.int32)) > 0

                def rbody(nd):
                    gg = plsc.load_gather(slab, [loc], mask=nd)
                    plsc.store_scatter(slab, [loc], jnp.maximum(gg, v),
                                       mask=nd)
                    gg2 = plsc.load_gather(slab, [loc], mask=nd)
                    return nd & (gg2 < v)

                lax.while_loop(rcond, rbody, need)
                return c3

            lax.fori_loop(0, nvec, vslow, 0)

    def _wait_pair(b):
        fbx, vbx, fs, vs = bufs[b]
        pltpu.make_async_copy(a_hbm.at[pl.ds(0, CB)], fbx, fs).wait()
        pltpu.make_async_copy(b_hbm.at[pl.ds(0, CB)], vbx, vs).wait()

    def _process_src(b, off, scnt):
        fbx, vbx, fs, vs = bufs[b]
        _process_chunk(fbx, vbx, scnt)
        nch = (scnt + CB - 1) // CB

        def extra(k, c):
            s2 = pl.ds(pl.multiple_of(off + k * CB, 8), CB)
            h1 = pltpu.async_copy(a_hbm.at[s2], fbx, fs)
            h2 = pltpu.async_copy(b_hbm.at[s2], vbx, vs)
            h1.wait()
            h2.wait()
            _process_chunk(fbx, vbx, scnt - k * CB)
            return c

        lax.fori_loop(1, nch, extra, 0)

    NPAIR = NW // 2
    nh.wait()
    m0 = _src_meta(0)
    _issue(0, m0[0])

    def pair_body(i, carry):
        off0, scnt0 = carry
        off0 = pl.multiple_of(off0, 8)
        off1, scnt1 = _src_meta(2 * i + 1)
        _issue(1, off1)
        _wait_pair(0)
        _process_src(0, off0, scnt0)
        t2 = jnp.minimum(2 * i + 2, NW - 1)
        off2, scnt2 = _src_meta(t2)

        @pl.when(i + 1 < NPAIR)
        def _():
            _issue(0, off2)

        _wait_pair(1)
        _process_src(1, off1, scnt1)
        return (off2, scnt2)

    lax.fori_loop(0, NPAIR, pair_body, m0)

    NG = SLAB // GC
    gbufs = ((gb0, ob0, gin0, gout0), (gb1, ob1, gin1, gout1))

    def start_g(gi):
        gb, _, gin, _ = gbufs[gi % 2]
        return pltpu.async_copy(grid_hbm.at[pl.ds(lo + gi * GC, GC)], gb, gin)

    gpend = start_g(0)
    opend = [None, None]
    for gi in range(NG):
        gnxt = start_g(gi + 1) if gi + 1 < NG else None
        gpend.wait()
        gb, ob, _, gout = gbufs[gi % 2]
        if opend[gi % 2] is not None:
            opend[gi % 2].wait()

        def gvec(j, c2, gb=gb, ob=ob, gi=gi):
            g = gb[pl.ds(j * L, L)]
            s = slab[pl.ds(gi * GC + j * L, L)]
            touched = s > -0.5
            new = jnp.where(touched,
                            jnp.maximum(g * EMA_D, jnp.maximum(s, 0.0)), g)
            ob[pl.ds(j * L, L)] = new
            return c2

        lax.fori_loop(0, GC // L, gvec, 0, unroll=4)
        opend[gi % 2] = pltpu.async_copy(
            ob, out_hbm.at[pl.ds(lo + gi * GC, GC)], gout)
        gpend = gnxt
    for h in opend:
        if h is not None:
            h.wait()


def _thresh_body(x_ref, o_ref):
    o_ref[...] = x_ref[...] > THRE


def _threshold(new_grid):
    return pl.pallas_call(
        _thresh_body,
        out_shape=jax.ShapeDtypeStruct((RES, RES, RES), jnp.bool_),
        grid=(4,),
        in_specs=[pl.BlockSpec((32, RES, RES), lambda i: (i, 0, 0))],
        out_specs=pl.BlockSpec((32, RES, RES), lambda i: (i, 0, 0)),
    )(new_grid)


def kernel(occ_val_grid, pts, val):
    grid_flat = occ_val_grid.reshape(-1)
    pts_t = pts.T
    a, b, m = _route_kernel(pts_t[0], pts_t[1], pts_t[2], val)
    neg = jnp.full((SLAB,), -1.0, jnp.float32)
    new_flat = _merge_ema_kernel(grid_flat, a, b, m, neg)
    new_grid = new_flat.reshape(RES, RES, RES)
    occ = _threshold(new_grid)
    return new_grid, occ
